# Initial kernel scaffold; baseline (speedup 1.0000x reference)
import jax
import jax.numpy as jnp
from jax.experimental import pallas as pl


def kernel(verts, edges, gc1_w0, gc1_b0, gc1_w1, gc1_b1, gc2_w0, gc2_b0, gc2_w1, gc2_b1, lin2_w, lin2_b, lin3_w, lin3_b):
    raise NotImplementedError("write your pallas kernel here")


# speedup vs baseline: 1.4797x; 1.4797x over previous
import functools

import jax
import jax.numpy as jnp
from jax.experimental import pallas as pl
from jax.experimental.pallas import tpu as pltpu

_HI = jax.lax.Precision.HIGHEST


def _rnd_bf16(a):
    return jax.lax.reduce_precision(a, exponent_bits=8, mantissa_bits=7)


def _mm_hi(a, b):
    return jnp.dot(a, b, precision=_HI, preferred_element_type=jnp.float32)


def _mm_bf(a, b):
    return jnp.dot(a.astype(jnp.bfloat16), b.astype(jnp.bfloat16),
                   preferred_element_type=jnp.float32)


def _mlp_body(m_ref, w2_ref, b2_ref, w3_ref, b3_ref, out_ref, h_ref):
    @pl.when(pl.program_id(0) == 0)
    def _():
        h = _mm_bf(m_ref[...], w2_ref[...].T)
        h_ref[...] = jax.nn.relu(h + b2_ref[...])
    y = _mm_bf(h_ref[...], w3_ref[...].T)
    out_ref[...] = jax.nn.sigmoid(y + b3_ref[...])


def _decoder(m, lin2_w, lin2_b, lin3_w, lin3_b):
    OUT = lin3_w.shape[0]
    TILE = 2048
    grid = OUT // TILE
    return pl.pallas_call(
        _mlp_body,
        grid=(grid,),
        in_specs=[
            pl.BlockSpec((1, 512), lambda i: (0, 0)),
            pl.BlockSpec((1024, 512), lambda i: (0, 0)),
            pl.BlockSpec((1, 1024), lambda i: (0, 0)),
            pl.BlockSpec((TILE, 1024), lambda i: (i, 0)),
            pl.BlockSpec((1, TILE), lambda i: (0, i)),
        ],
        out_specs=pl.BlockSpec((1, TILE), lambda i: (0, i)),
        out_shape=jax.ShapeDtypeStruct((1, OUT), jnp.float32),
        scratch_shapes=[pltpu.VMEM((1, 1024), jnp.float32)],
    )(m.reshape(1, 512), lin2_w, lin2_b.reshape(1, 1024), lin3_w,
      lin3_b.reshape(1, OUT))


def kernel(verts, edges, gc1_w0, gc1_b0, gc1_w1, gc1_b1,
           gc2_w0, gc2_b0, gc2_w1, gc2_b1,
           lin2_w, lin2_b, lin3_w, lin3_b):
    n = verts.shape[0]
    e0, e1 = edges[:, 0], edges[:, 1]

    vx = jnp.concatenate([verts, jnp.ones((n, 1), jnp.float32)], axis=1)
    agg4 = jnp.zeros((n, 4), jnp.float32)
    agg4 = agg4.at[e0].add(vx[e1]).at[e1].add(vx[e0])
    aggv, deg = agg4[:, :3], agg4[:, 3:4]

    x1 = jax.nn.relu(_mm_hi(verts, gc1_w0.T) + gc1_b0 + _mm_hi(aggv, gc1_w1.T)
                     + deg * gc1_b1)
    x1r = _rnd_bf16(x1)

    aggx = jnp.zeros_like(x1r)
    aggx = aggx.at[e0].add(x1r[e1]).at[e1].add(x1r[e0])
    x2 = jax.nn.relu(_mm_bf(x1r, gc2_w0.T) + gc2_b0
                     + _mm_hi(aggx, _rnd_bf16(gc2_w1).T) + deg * gc2_b1)

    m = jnp.max(x2, axis=0)
    img = _decoder(m, lin2_w, lin2_b, lin3_w, lin3_b)
    img = img.reshape(1, 3, 128, 128)
    img = jnp.repeat(jnp.repeat(img, 4, axis=2), 4, axis=3)
    return img
